# speedup vs baseline: 9.3443x; 9.3443x over previous
import functools

import jax
import jax.numpy as jnp
from jax import lax
from jax.experimental import pallas as pl
from jax.experimental.pallas import tpu as pltpu
from jax.experimental.pallas import tpu_sc as plsc

N = 10000
NP = 10240
E = 320000
IN = 128
H = 256
OUT = 128
HALF = 128

NSUB = 16
NCORE = 2
W = 128
EP = 4096 * 79
NWIN = EP // (NSUB * W)
NWIN_DEG = NWIN // NCORE
RPS = NP // NSUB

_mesh = plsc.VectorSubcoreMesh(core_axis_name="c", subcore_axis_name="s")


@functools.partial(
    pl.kernel,
    mesh=_mesh,
    out_type=jax.ShapeDtypeStruct((NCORE, NP), jnp.float32),
    scratch_types=[
        pltpu.VMEM((W,), jnp.int32),
        pltpu.VMEM((W,), jnp.float32),
        pltpu.VMEM((RPS,), jnp.float32),
        pltpu.VMEM_SHARED((NP,), jnp.float32),
    ],
)
def _deg_kernel(dst_hbm, deg_hbm, idx_v, ones_v, init_v, acc_sh):
    c = lax.axis_index("c")
    s = lax.axis_index("s")
    for k in range(W // 16):
        ones_v[pl.ds(k * 16, 16)] = jnp.full((16,), 1.0, jnp.float32)
    for k in range(RPS // 16):
        init_v[pl.ds(k * 16, 16)] = jnp.full((16,), 0.5, jnp.float32)
    pltpu.sync_copy(init_v, acc_sh.at[pl.ds(s * RPS, RPS)])
    plsc.subcore_barrier()

    def body(j, carry):
        pltpu.sync_copy(dst_hbm.at[s, c * NWIN_DEG + j], idx_v)
        pltpu.sync_copy(ones_v, acc_sh.at[idx_v], add=True)
        return carry

    lax.fori_loop(0, NWIN_DEG, body, 0)
    plsc.subcore_barrier()
    pltpu.sync_copy(acc_sh.at[pl.ds(s * RPS, RPS)],
                    deg_hbm.at[c, pl.ds(s * RPS, RPS)])


@functools.partial(
    pl.kernel,
    mesh=_mesh,
    out_type=jax.ShapeDtypeStruct((NCORE, NP, HALF), jnp.float32),
    scratch_types=[
        pltpu.VMEM((W,), jnp.int32),
        pltpu.VMEM((W,), jnp.int32),
        pltpu.VMEM((W,), jnp.int32),
        pltpu.VMEM((W, HALF), jnp.float32),
        pltpu.VMEM_SHARED((NP, HALF), jnp.float32),
        pltpu.SemaphoreType.DMA,
    ],
)
def _msg_kernel(g_hbm, src_hbm, dst_hbm, m_hbm,
                idx_s, idx_g, idx_d, rows_v, acc_sh, sem):
    c = lax.axis_index("c")
    s = lax.axis_index("s")
    off = c * NP
    r0 = s * RPS
    pltpu.sync_copy(g_hbm.at[pl.ds(off + r0, RPS)], acc_sh.at[pl.ds(r0, RPS)])
    plsc.subcore_barrier()

    def body(j, carry):
        pltpu.sync_copy(src_hbm.at[s, j], idx_s)
        for k in range(W // 16):
            sl = pl.ds(k * 16, 16)
            idx_g[sl] = idx_s[sl] + off
        pltpu.async_copy(g_hbm.at[idx_g], rows_v, sem).wait()
        pltpu.sync_copy(dst_hbm.at[s, j], idx_d)
        pltpu.sync_copy(rows_v, acc_sh.at[idx_d], add=True)
        return carry

    lax.fori_loop(0, NWIN, body, 0)
    plsc.subcore_barrier()
    pltpu.sync_copy(acc_sh.at[pl.ds(r0, RPS)],
                    m_hbm.at[c, pl.ds(r0, RPS)])


BM = 1024
GRID = NP // BM


def _tc_first_body(x_ref, mean_ref, std_ref, w_ref, deg_ref, g_ref):
    h0 = (x_ref[...] - mean_ref[...]) / std_ref[...]
    p = jnp.dot(h0, w_ref[...], preferred_element_type=jnp.float32)
    dinv = lax.rsqrt(deg_ref[0] + deg_ref[1])
    g = p * dinv
    g_ref[0] = g[:, :HALF]
    g_ref[1] = g[:, HALF:]


def _tc_mid_body(m_ref, deg_ref, b_ref, w_ref, g_ref):
    dinv = lax.rsqrt(deg_ref[0] + deg_ref[1])
    mb = jnp.concatenate([m_ref[0], m_ref[1]], axis=1)
    h = jnp.maximum(mb * dinv + b_ref[...], 0.0)
    p = jnp.dot(h, w_ref[...], preferred_element_type=jnp.float32)
    g = p * dinv
    g_ref[0] = g[:, :HALF]
    g_ref[1] = g[:, HALF:]


def _tc_final_body(m_ref, deg_ref, b_ref, wl_ref, bl_ref, ostd_ref,
                   omean_ref, mask_ref, pq_ref, o_ref):
    dinv = lax.rsqrt(deg_ref[0] + deg_ref[1])
    mb = jnp.concatenate([m_ref[0], m_ref[1]], axis=1)
    h = mb * dinv + b_ref[...]
    o = jnp.dot(h, wl_ref[...], preferred_element_type=jnp.float32) + bl_ref[...]
    o = o * ostd_ref[...] + omean_ref[...]
    o_ref[...] = jnp.where(mask_ref[...], pq_ref[...], o)


def _row_spec(width):
    return pl.BlockSpec((BM, width), lambda i: (i, 0))


def _full_spec(shape):
    nd = len(shape)
    return pl.BlockSpec(shape, lambda i, _n=nd: (0,) * _n)


_deg_spec = pl.BlockSpec((NCORE, BM, 1), lambda i: (0, i, 0))
_gm_spec = pl.BlockSpec((NCORE, BM, HALF), lambda i: (0, i, 0))


def _tc_first(x_pad, in_mean, in_std, w1, deg3):
    return pl.pallas_call(
        _tc_first_body,
        grid=(GRID,),
        in_specs=[_row_spec(IN), _full_spec((1, IN)), _full_spec((1, IN)),
                  _full_spec((IN, H)), _deg_spec],
        out_specs=_gm_spec,
        out_shape=jax.ShapeDtypeStruct((NCORE, NP, HALF), jnp.float32),
    )(x_pad, in_mean.reshape(1, IN), in_std.reshape(1, IN), w1, deg3)


def _tc_mid(m, deg3, b, w):
    return pl.pallas_call(
        _tc_mid_body,
        grid=(GRID,),
        in_specs=[_gm_spec, _deg_spec, _full_spec((1, H)), _full_spec((H, H))],
        out_specs=_gm_spec,
        out_shape=jax.ShapeDtypeStruct((NCORE, NP, HALF), jnp.float32),
    )(m, deg3, b.reshape(1, H), w)


def _tc_final(m, deg3, b3, wlin, blin, out_std, out_mean, mask_pad, pq_pad):
    return pl.pallas_call(
        _tc_final_body,
        grid=(GRID,),
        in_specs=[_gm_spec, _deg_spec, _full_spec((1, H)), _full_spec((H, OUT)),
                  _full_spec((1, OUT)), _full_spec((1, OUT)), _full_spec((1, OUT)),
                  _row_spec(OUT), _row_spec(OUT)],
        out_specs=_row_spec(OUT),
        out_shape=jax.ShapeDtypeStruct((NP, OUT), jnp.float32),
    )(m, deg3, b3.reshape(1, H), wlin, blin.reshape(1, OUT),
      out_std.reshape(1, OUT), out_mean.reshape(1, OUT), mask_pad, pq_pad)


def kernel(x, edge_index, PQVA_mask, PQVA_matrix, in_mean, in_std,
           out_mean, out_std, W1, b1, W2, b2, W3, b3, Wlin, blin):
    pad = EP - E
    pad_idx = N + (jnp.arange(pad, dtype=jnp.int32) % (NP - N))
    src3 = jnp.concatenate([edge_index[0], pad_idx]).reshape(NSUB, NWIN, W)
    dst3 = jnp.concatenate([edge_index[1], pad_idx]).reshape(NSUB, NWIN, W)
    x_pad = jnp.pad(x, ((0, NP - N), (0, 0)))
    mask_pad = jnp.pad(PQVA_mask, ((0, NP - N), (0, 0)))
    pq_pad = jnp.pad(PQVA_matrix, ((0, NP - N), (0, 0)))

    deg3 = _deg_kernel(dst3).reshape(NCORE, NP, 1)

    g1 = _tc_first(x_pad, in_mean, in_std, W1, deg3)
    m1 = _msg_kernel(g1.reshape(NCORE * NP, HALF), src3, dst3)
    g2 = _tc_mid(m1, deg3, b1, W2)
    m2 = _msg_kernel(g2.reshape(NCORE * NP, HALF), src3, dst3)
    g3 = _tc_mid(m2, deg3, b2, W3)
    m3 = _msg_kernel(g3.reshape(NCORE * NP, HALF), src3, dst3)
    out = _tc_final(m3, deg3, b3, Wlin, blin, out_std, out_mean,
                    mask_pad, pq_pad)
    return out[:N]


# speedup vs baseline: 20.0722x; 2.1481x over previous
import functools

import jax
import jax.numpy as jnp
from jax import lax
from jax.experimental import pallas as pl
from jax.experimental.pallas import tpu as pltpu
from jax.experimental.pallas import tpu_sc as plsc

N = 10000
NP = 10240
E = 320000
IN = 128
H = 256
OUT = 128
HALF = 128

NSUB = 16
NCORE = 2
W = 128
NWIN = 160
EP = NSUB * NWIN * W
NWIN_DEG = NWIN // NCORE
RPS = NP // NSUB
CH = 1
NG = 2 * CH
NCHUNK = NWIN // CH

_mesh = plsc.VectorSubcoreMesh(core_axis_name="c", subcore_axis_name="s")


WD = 128
ND = EP // (NSUB * NCORE * WD)


@functools.partial(
    pl.kernel,
    mesh=_mesh,
    out_type=jax.ShapeDtypeStruct((NCORE, NP), jnp.float32),
    scratch_types=[
        pltpu.VMEM((2, WD), jnp.int32),
        pltpu.VMEM((WD,), jnp.float32),
        pltpu.VMEM((RPS,), jnp.float32),
        pltpu.VMEM_SHARED((NP,), jnp.float32),
        pltpu.SemaphoreType.DMA,
        pltpu.SemaphoreType.DMA,
        pltpu.SemaphoreType.DMA,
        pltpu.SemaphoreType.DMA,
    ],
)
def _deg_kernel(dstd_hbm, deg_hbm, idx_v, ones_v, init_v, acc_sh,
                isemA, isemB, ssemA, ssemB):
    c = lax.axis_index("c")
    s = lax.axis_index("s")
    w = s * NCORE + c
    for k in range(WD // 16):
        ones_v[pl.ds(k * 16, 16)] = jnp.full((16,), 1.0, jnp.float32)
    for k in range(RPS // 16):
        init_v[pl.ds(k * 16, 16)] = jnp.full((16,), 0.5, jnp.float32)
    pltpu.sync_copy(init_v, acc_sh.at[pl.ds(s * RPS, RPS)])
    plsc.subcore_barrier()

    isems = [isemA, isemB]
    ssems = [ssemA, ssemB]

    def i_start(j, p):
        pltpu.async_copy(dstd_hbm.at[w, j], idx_v.at[p], isems[p])

    def i_wait(j, p):
        pltpu.make_async_copy(dstd_hbm.at[w, j], idx_v.at[p],
                              isems[p]).wait()

    def s_start(p):
        pltpu.async_copy(ones_v, acc_sh.at[idx_v.at[p]], ssems[p], add=True)

    def s_wait(p):
        pltpu.make_async_copy(ones_v, acc_sh.at[idx_v.at[0]],
                              ssems[p]).wait()

    i_start(0, 0)
    i_start(1, 1)

    def body(t, carry):
        j0 = 2 * t
        for p in range(2):
            i_wait(j0 + p, p)
            s_start(p)

        @pl.when(t + 1 < ND // 2)
        def _():
            for p in range(2):
                s_wait(p)
                i_start(j0 + 2 + p, p)
        return carry

    lax.fori_loop(0, ND // 2, body, 0)
    for p in range(2):
        s_wait(p)
    plsc.subcore_barrier()
    pltpu.sync_copy(acc_sh.at[pl.ds(s * RPS, RPS)],
                    deg_hbm.at[c, pl.ds(s * RPS, RPS)])


@functools.partial(
    pl.kernel,
    mesh=_mesh,
    out_type=jax.ShapeDtypeStruct((NCORE, NP, HALF), jnp.float32),
    scratch_types=[
        pltpu.VMEM((CH, 2, W), jnp.int32),
        pltpu.VMEM((CH, 2, W), jnp.int32),
        pltpu.VMEM((CH, W), jnp.int32),
        pltpu.VMEM((CH, W), jnp.int32),
        pltpu.VMEM((NG, W, HALF), jnp.float32),
        pltpu.VMEM_SHARED((NP, HALF), jnp.float32),
        pltpu.SemaphoreType.DMA,
        pltpu.SemaphoreType.DMA,
        pltpu.SemaphoreType.DMA,
        pltpu.SemaphoreType.DMA,
        pltpu.SemaphoreType.DMA,
        pltpu.SemaphoreType.DMA,
    ],
)
def _msg_kernel(g_hbm, idx_hbm, m_hbm,
                ibuf0, ibuf1, sbuf0, sbuf1, rows_v, acc_sh,
                gsem0, gsem1, ssem0, ssem1, isem0, isem1):
    c = lax.axis_index("c")
    s = lax.axis_index("s")
    r0 = s * RPS
    pltpu.sync_copy(g_hbm.at[pl.ds(c * NP + r0, RPS)],
                    acc_sh.at[pl.ds(r0, RPS)])
    plsc.subcore_barrier()

    def i_start(j0, ibuf, isem):
        pltpu.async_copy(idx_hbm.at[c, s, pl.ds(j0, CH)], ibuf, isem)

    def i_wait(j0, ibuf, isem):
        pltpu.make_async_copy(idx_hbm.at[c, s, pl.ds(j0, CH)], ibuf,
                              isem).wait()

    def g_start(i, buf, ibuf, sem):
        pltpu.async_copy(g_hbm.at[ibuf.at[i, 0]], rows_v.at[buf], sem)

    def g_wait(i, buf, ibuf, sem):
        pltpu.make_async_copy(g_hbm.at[ibuf.at[i, 0]], rows_v.at[buf],
                              sem).wait()

    def dst_copy(ibuf, sbuf):
        for i in range(CH):
            for k in range(W // 16):
                sl = pl.ds(k * 16, 16)
                sbuf[i, sl] = ibuf[i, 1, sl]

    def s_start(i, buf, sbuf, sem):
        pltpu.async_copy(rows_v.at[buf], acc_sh.at[sbuf.at[i]], sem,
                         add=True)

    def s_wait(i, buf, sbuf, sem):
        pltpu.make_async_copy(rows_v.at[buf], acc_sh.at[sbuf.at[i]],
                              sem).wait()

    pltpu.sync_copy(idx_hbm.at[c, s, pl.ds(0, CH)], ibuf0)
    for i in range(CH):
        g_start(i, i, ibuf0, gsem0)
    i_start(CH, ibuf1, isem1)

    def pair(t, carry):
        a0 = (2 * t) * CH
        b0 = (2 * t + 1) * CH

        @pl.when(t > 0)
        def _():
            for i in range(CH):
                s_wait(i, CH + i, sbuf1, ssem1)
        i_wait(b0, ibuf1, isem1)
        for i in range(CH):
            g_start(i, CH + i, ibuf1, gsem1)
        for i in range(CH):
            g_wait(i, i, ibuf0, gsem0)
        dst_copy(ibuf0, sbuf0)
        for i in range(CH):
            s_start(i, i, sbuf0, ssem0)

        @pl.when(t + 1 < NCHUNK // 2)
        def _():
            i_start(a0 + 2 * CH, ibuf0, isem0)
        for i in range(CH):
            g_wait(i, CH + i, ibuf1, gsem1)
        dst_copy(ibuf1, sbuf1)
        for i in range(CH):
            s_wait(i, i, sbuf0, ssem0)

        @pl.when(t + 1 < NCHUNK // 2)
        def _():
            i_wait((2 * t + 2) * CH, ibuf0, isem0)
            for i in range(CH):
                g_start(i, i, ibuf0, gsem0)
            i_start((2 * t + 3) * CH, ibuf1, isem1)
        for i in range(CH):
            s_start(i, CH + i, sbuf1, ssem1)
        return carry

    lax.fori_loop(0, NCHUNK // 2, pair, 0)
    for i in range(CH):
        s_wait(i, CH + i, sbuf1, ssem1)
    plsc.subcore_barrier()
    pltpu.sync_copy(acc_sh.at[pl.ds(r0, RPS)],
                    m_hbm.at[c, pl.ds(r0, RPS)])


BM = 1000
GRID = N // BM


def _tc_first_body(x_ref, mean_ref, std_ref, w_ref, deg_ref, g_ref):
    h0 = (x_ref[...] - mean_ref[...]) / std_ref[...]
    p = jnp.dot(h0, w_ref[...], preferred_element_type=jnp.float32)
    dinv = lax.rsqrt(deg_ref[0] + deg_ref[1])
    g = p * dinv
    g_ref[0] = g[:, :HALF]
    g_ref[1] = g[:, HALF:]


def _tc_mid_body(m_ref, deg_ref, b_ref, w_ref, g_ref):
    dinv = lax.rsqrt(deg_ref[0] + deg_ref[1])
    mb = jnp.concatenate([m_ref[0], m_ref[1]], axis=1)
    h = jnp.maximum(mb * dinv + b_ref[...], 0.0)
    p = jnp.dot(h, w_ref[...], preferred_element_type=jnp.float32)
    g = p * dinv
    g_ref[0] = g[:, :HALF]
    g_ref[1] = g[:, HALF:]


def _tc_final_body(m_ref, deg_ref, b_ref, wl_ref, bl_ref, ostd_ref,
                   omean_ref, mask_ref, pq_ref, o_ref):
    dinv = lax.rsqrt(deg_ref[0] + deg_ref[1])
    mb = jnp.concatenate([m_ref[0], m_ref[1]], axis=1)
    h = mb * dinv + b_ref[...]
    o = jnp.dot(h, wl_ref[...], preferred_element_type=jnp.float32) + bl_ref[...]
    o = o * ostd_ref[...] + omean_ref[...]
    o_ref[...] = jnp.where(mask_ref[...], pq_ref[...], o)


def _row_spec(width):
    return pl.BlockSpec((BM, width), lambda i: (i, 0))


def _full_spec(shape):
    nd = len(shape)
    return pl.BlockSpec(shape, lambda i, _n=nd: (0,) * _n)


_deg_spec = pl.BlockSpec((NCORE, BM, 1), lambda i: (0, i, 0))
_gm_spec = pl.BlockSpec((NCORE, BM, HALF), lambda i: (0, i, 0))


def _tc_first(x_pad, in_mean, in_std, w1, deg3):
    return pl.pallas_call(
        _tc_first_body,
        grid=(GRID,),
        in_specs=[_row_spec(IN), _full_spec((1, IN)), _full_spec((1, IN)),
                  _full_spec((IN, H)), _deg_spec],
        out_specs=_gm_spec,
        out_shape=jax.ShapeDtypeStruct((NCORE, NP, HALF), jnp.float32),
    )(x_pad, in_mean.reshape(1, IN), in_std.reshape(1, IN), w1, deg3)


def _tc_mid(m, deg3, b, w):
    return pl.pallas_call(
        _tc_mid_body,
        grid=(GRID,),
        in_specs=[_gm_spec, _deg_spec, _full_spec((1, H)), _full_spec((H, H))],
        out_specs=_gm_spec,
        out_shape=jax.ShapeDtypeStruct((NCORE, NP, HALF), jnp.float32),
    )(m, deg3, b.reshape(1, H), w)


def _tc_final(m, deg3, b3, wlin, blin, out_std, out_mean, mask_pad, pq_pad):
    return pl.pallas_call(
        _tc_final_body,
        grid=(GRID,),
        in_specs=[_gm_spec, _deg_spec, _full_spec((1, H)), _full_spec((H, OUT)),
                  _full_spec((1, OUT)), _full_spec((1, OUT)), _full_spec((1, OUT)),
                  _row_spec(OUT), _row_spec(OUT)],
        out_specs=_row_spec(OUT),
        out_shape=jax.ShapeDtypeStruct((N, OUT), jnp.float32),
    )(m, deg3, b3.reshape(1, H), wlin, blin.reshape(1, OUT),
      out_std.reshape(1, OUT), out_mean.reshape(1, OUT), mask_pad, pq_pad)


def kernel(x, edge_index, PQVA_mask, PQVA_matrix, in_mean, in_std,
           out_mean, out_std, W1, b1, W2, b2, W3, b3, Wlin, blin):
    pad = EP - E
    pad_idx = N + (jnp.arange(pad, dtype=jnp.int32) % (NP - N))
    srcp = jnp.concatenate([edge_index[0], pad_idx])
    dstp = jnp.concatenate([edge_index[1], pad_idx])
    src3 = srcp.reshape(NSUB, NWIN, W)
    dst3 = dstp.reshape(NSUB, NWIN, W)
    idx_hbm = jnp.stack([
        jnp.stack([src3, dst3], axis=2),
        jnp.stack([src3 + NP, dst3], axis=2),
    ])
    dstd = dstp.reshape(NSUB * NCORE, ND, WD)

    deg3 = _deg_kernel(dst3).reshape(NCORE, NP, 1)

    g1 = _tc_first(x, in_mean, in_std, W1, deg3)
    m1 = _msg_kernel(g1.reshape(NCORE * NP, HALF), src3b, dst3)
    g2 = _tc_mid(m1, deg3, b1, W2)
    m2 = _msg_kernel(g2.reshape(NCORE * NP, HALF), src3b, dst3)
    g3 = _tc_mid(m2, deg3, b2, W3)
    m3 = _msg_kernel(g3.reshape(NCORE * NP, HALF), src3b, dst3)
    return _tc_final(m3, deg3, b3, Wlin, blin, out_std, out_mean,
                     PQVA_mask, PQVA_matrix)


# speedup vs baseline: 20.1183x; 1.0023x over previous
import functools

import jax
import jax.numpy as jnp
from jax import lax
from jax.experimental import pallas as pl
from jax.experimental.pallas import tpu as pltpu
from jax.experimental.pallas import tpu_sc as plsc

N = 10000
NP = 10240
E = 320000
IN = 128
H = 256
OUT = 128
HALF = 128

NSUB = 16
NCORE = 2
W = 128
NWIN = 160
EP = NSUB * NWIN * W
NWIN_DEG = NWIN // NCORE
RPS = NP // NSUB
CH = 1
NG = 2 * CH
NCHUNK = NWIN // CH

_mesh = plsc.VectorSubcoreMesh(core_axis_name="c", subcore_axis_name="s")


WD = 128
ND = EP // (NSUB * NCORE * WD)


@functools.partial(
    pl.kernel,
    mesh=_mesh,
    out_type=jax.ShapeDtypeStruct((NCORE, NP), jnp.float32),
    scratch_types=[
        pltpu.VMEM((2, WD), jnp.int32),
        pltpu.VMEM((WD,), jnp.float32),
        pltpu.VMEM((RPS,), jnp.float32),
        pltpu.VMEM_SHARED((NP,), jnp.float32),
        pltpu.SemaphoreType.DMA,
        pltpu.SemaphoreType.DMA,
        pltpu.SemaphoreType.DMA,
        pltpu.SemaphoreType.DMA,
    ],
)
def _deg_kernel(dstd_hbm, deg_hbm, idx_v, ones_v, init_v, acc_sh,
                isemA, isemB, ssemA, ssemB):
    c = lax.axis_index("c")
    s = lax.axis_index("s")
    w = s * NCORE + c
    for k in range(WD // 16):
        ones_v[pl.ds(k * 16, 16)] = jnp.full((16,), 1.0, jnp.float32)
    for k in range(RPS // 16):
        init_v[pl.ds(k * 16, 16)] = jnp.full((16,), 0.5, jnp.float32)
    isems = [isemA, isemB]
    ssems = [ssemA, ssemB]

    def i_start(j, p):
        pltpu.async_copy(dstd_hbm.at[w, j], idx_v.at[p], isems[p])

    def i_wait(j, p):
        pltpu.make_async_copy(dstd_hbm.at[w, j], idx_v.at[p],
                              isems[p]).wait()

    def s_start(p):
        pltpu.async_copy(ones_v, acc_sh.at[idx_v.at[p]], ssems[p], add=True)

    def s_wait(p):
        pltpu.make_async_copy(ones_v, acc_sh.at[idx_v.at[0]],
                              ssems[p]).wait()

    i_start(0, 0)
    i_start(1, 1)
    pltpu.sync_copy(init_v, acc_sh.at[pl.ds(s * RPS, RPS)])
    plsc.subcore_barrier()

    def body(t, carry):
        j0 = 2 * t
        for p in range(2):
            i_wait(j0 + p, p)
            s_start(p)

        @pl.when(t + 1 < ND // 2)
        def _():
            for p in range(2):
                s_wait(p)
                i_start(j0 + 2 + p, p)
        return carry

    lax.fori_loop(0, ND // 2, body, 0)
    for p in range(2):
        s_wait(p)
    plsc.subcore_barrier()
    pltpu.sync_copy(acc_sh.at[pl.ds(s * RPS, RPS)],
                    deg_hbm.at[c, pl.ds(s * RPS, RPS)])


@functools.partial(
    pl.kernel,
    mesh=_mesh,
    out_type=jax.ShapeDtypeStruct((NCORE, NP, HALF), jnp.float32),
    scratch_types=[
        pltpu.VMEM((CH, 2, W), jnp.int32),
        pltpu.VMEM((CH, 2, W), jnp.int32),
        pltpu.VMEM((CH, W), jnp.int32),
        pltpu.VMEM((CH, W), jnp.int32),
        pltpu.VMEM((NG, W, HALF), jnp.float32),
        pltpu.VMEM_SHARED((NP, HALF), jnp.float32),
        pltpu.SemaphoreType.DMA,
        pltpu.SemaphoreType.DMA,
        pltpu.SemaphoreType.DMA,
        pltpu.SemaphoreType.DMA,
        pltpu.SemaphoreType.DMA,
        pltpu.SemaphoreType.DMA,
    ],
)
def _msg_kernel(g_hbm, idx_hbm, m_hbm,
                ibuf0, ibuf1, sbuf0, sbuf1, rows_v, acc_sh,
                gsem0, gsem1, ssem0, ssem1, isem0, isem1):
    c = lax.axis_index("c")
    s = lax.axis_index("s")
    r0 = s * RPS

    def i_start(j0, ibuf, isem):
        pltpu.async_copy(idx_hbm.at[c, s, pl.ds(j0, CH)], ibuf, isem)

    def i_wait(j0, ibuf, isem):
        pltpu.make_async_copy(idx_hbm.at[c, s, pl.ds(j0, CH)], ibuf,
                              isem).wait()

    def g_start(i, buf, ibuf, sem):
        pltpu.async_copy(g_hbm.at[ibuf.at[i, 0]], rows_v.at[buf], sem)

    def g_wait(i, buf, ibuf, sem):
        pltpu.make_async_copy(g_hbm.at[ibuf.at[i, 0]], rows_v.at[buf],
                              sem).wait()

    def dst_copy(ibuf, sbuf):
        for i in range(CH):
            for k in range(W // 16):
                sl = pl.ds(k * 16, 16)
                sbuf[i, sl] = ibuf[i, 1, sl]

    def s_start(i, buf, sbuf, sem):
        pltpu.async_copy(rows_v.at[buf], acc_sh.at[sbuf.at[i]], sem,
                         add=True)

    def s_wait(i, buf, sbuf, sem):
        pltpu.make_async_copy(rows_v.at[buf], acc_sh.at[sbuf.at[i]],
                              sem).wait()

    i_start(0, ibuf0, isem0)
    pltpu.sync_copy(g_hbm.at[pl.ds(c * NP + r0, RPS)],
                    acc_sh.at[pl.ds(r0, RPS)])
    i_wait(0, ibuf0, isem0)
    for i in range(CH):
        g_start(i, i, ibuf0, gsem0)
    i_start(CH, ibuf1, isem1)
    plsc.subcore_barrier()

    def pair(t, carry):
        a0 = (2 * t) * CH
        b0 = (2 * t + 1) * CH

        @pl.when(t > 0)
        def _():
            for i in range(CH):
                s_wait(i, CH + i, sbuf1, ssem1)
        i_wait(b0, ibuf1, isem1)
        for i in range(CH):
            g_start(i, CH + i, ibuf1, gsem1)
        for i in range(CH):
            g_wait(i, i, ibuf0, gsem0)
        dst_copy(ibuf0, sbuf0)
        for i in range(CH):
            s_start(i, i, sbuf0, ssem0)

        @pl.when(t + 1 < NCHUNK // 2)
        def _():
            i_start(a0 + 2 * CH, ibuf0, isem0)
        for i in range(CH):
            g_wait(i, CH + i, ibuf1, gsem1)
        dst_copy(ibuf1, sbuf1)
        for i in range(CH):
            s_wait(i, i, sbuf0, ssem0)

        @pl.when(t + 1 < NCHUNK // 2)
        def _():
            i_wait((2 * t + 2) * CH, ibuf0, isem0)
            for i in range(CH):
                g_start(i, i, ibuf0, gsem0)
            i_start((2 * t + 3) * CH, ibuf1, isem1)
        for i in range(CH):
            s_start(i, CH + i, sbuf1, ssem1)
        return carry

    lax.fori_loop(0, NCHUNK // 2, pair, 0)
    for i in range(CH):
        s_wait(i, CH + i, sbuf1, ssem1)
    plsc.subcore_barrier()
    pltpu.sync_copy(acc_sh.at[pl.ds(r0, RPS)],
                    m_hbm.at[c, pl.ds(r0, RPS)])


BM = 1000
GRID = N // BM


def _tc_first_body(x_ref, mean_ref, std_ref, w_ref, deg_ref, g_ref):
    h0 = (x_ref[...] - mean_ref[...]) / std_ref[...]
    p = jnp.dot(h0, w_ref[...], preferred_element_type=jnp.float32)
    dinv = lax.rsqrt(deg_ref[0] + deg_ref[1])
    g = p * dinv
    g_ref[0] = g[:, :HALF]
    g_ref[1] = g[:, HALF:]


def _tc_mid_body(m_ref, deg_ref, b_ref, w_ref, g_ref):
    dinv = lax.rsqrt(deg_ref[0] + deg_ref[1])
    mb = jnp.concatenate([m_ref[0], m_ref[1]], axis=1)
    h = jnp.maximum(mb * dinv + b_ref[...], 0.0)
    p = jnp.dot(h, w_ref[...], preferred_element_type=jnp.float32)
    g = p * dinv
    g_ref[0] = g[:, :HALF]
    g_ref[1] = g[:, HALF:]


def _tc_final_body(m_ref, deg_ref, b_ref, wl_ref, bl_ref, ostd_ref,
                   omean_ref, mask_ref, pq_ref, o_ref):
    dinv = lax.rsqrt(deg_ref[0] + deg_ref[1])
    mb = jnp.concatenate([m_ref[0], m_ref[1]], axis=1)
    h = mb * dinv + b_ref[...]
    o = jnp.dot(h, wl_ref[...], preferred_element_type=jnp.float32) + bl_ref[...]
    o = o * ostd_ref[...] + omean_ref[...]
    o_ref[...] = jnp.where(mask_ref[...], pq_ref[...], o)


def _row_spec(width):
    return pl.BlockSpec((BM, width), lambda i: (i, 0))


def _full_spec(shape):
    nd = len(shape)
    return pl.BlockSpec(shape, lambda i, _n=nd: (0,) * _n)


_deg_spec = pl.BlockSpec((NCORE, BM, 1), lambda i: (0, i, 0))
_gm_spec = pl.BlockSpec((NCORE, BM, HALF), lambda i: (0, i, 0))


def _tc_first(x_pad, in_mean, in_std, w1, deg3):
    return pl.pallas_call(
        _tc_first_body,
        grid=(GRID,),
        in_specs=[_row_spec(IN), _full_spec((1, IN)), _full_spec((1, IN)),
                  _full_spec((IN, H)), _deg_spec],
        out_specs=_gm_spec,
        out_shape=jax.ShapeDtypeStruct((NCORE, NP, HALF), jnp.float32),
    )(x_pad, in_mean.reshape(1, IN), in_std.reshape(1, IN), w1, deg3)


def _tc_mid(m, deg3, b, w):
    return pl.pallas_call(
        _tc_mid_body,
        grid=(GRID,),
        in_specs=[_gm_spec, _deg_spec, _full_spec((1, H)), _full_spec((H, H))],
        out_specs=_gm_spec,
        out_shape=jax.ShapeDtypeStruct((NCORE, NP, HALF), jnp.float32),
    )(m, deg3, b.reshape(1, H), w)


def _tc_final(m, deg3, b3, wlin, blin, out_std, out_mean, mask_pad, pq_pad):
    return pl.pallas_call(
        _tc_final_body,
        grid=(GRID,),
        in_specs=[_gm_spec, _deg_spec, _full_spec((1, H)), _full_spec((H, OUT)),
                  _full_spec((1, OUT)), _full_spec((1, OUT)), _full_spec((1, OUT)),
                  _row_spec(OUT), _row_spec(OUT)],
        out_specs=_row_spec(OUT),
        out_shape=jax.ShapeDtypeStruct((N, OUT), jnp.float32),
    )(m, deg3, b3.reshape(1, H), wlin, blin.reshape(1, OUT),
      out_std.reshape(1, OUT), out_mean.reshape(1, OUT), mask_pad, pq_pad)


def kernel(x, edge_index, PQVA_mask, PQVA_matrix, in_mean, in_std,
           out_mean, out_std, W1, b1, W2, b2, W3, b3, Wlin, blin):
    pad = EP - E
    pad_idx = N + (jnp.arange(pad, dtype=jnp.int32) % (NP - N))
    srcp = jnp.concatenate([edge_index[0], pad_idx])
    dstp = jnp.concatenate([edge_index[1], pad_idx])
    src3 = srcp.reshape(NSUB, NWIN, W)
    dst3 = dstp.reshape(NSUB, NWIN, W)
    idx_hbm = jnp.stack([
        jnp.stack([src3, dst3], axis=2),
        jnp.stack([src3 + NP, dst3], axis=2),
    ])
    dstd = dstp.reshape(NSUB * NCORE, ND, WD)

    deg3 = _deg_kernel(dst3).reshape(NCORE, NP, 1)

    g1 = _tc_first(x, in_mean, in_std, W1, deg3)
    m1 = _msg_kernel(g1.reshape(NCORE * NP, HALF), src3b, dst3)
    g2 = _tc_mid(m1, deg3, b1, W2)
    m2 = _msg_kernel(g2.reshape(NCORE * NP, HALF), src3b, dst3)
    g3 = _tc_mid(m2, deg3, b2, W3)
    m3 = _msg_kernel(g3.reshape(NCORE * NP, HALF), src3b, dst3)
    return _tc_final(m3, deg3, b3, Wlin, blin, out_std, out_mean,
                     PQVA_mask, PQVA_matrix)


# speedup vs baseline: 20.1969x; 1.0039x over previous
import functools

import jax
import jax.numpy as jnp
from jax import lax
from jax.experimental import pallas as pl
from jax.experimental.pallas import tpu as pltpu
from jax.experimental.pallas import tpu_sc as plsc

N = 10000
NP = 10240
E = 320000
IN = 128
H = 256
OUT = 128
HALF = 128

NSUB = 16
NCORE = 2
W = 128
NWIN = 160
EP = NSUB * NWIN * W
NWIN_DEG = NWIN // NCORE
RPS = NP // NSUB
CH = 1
NG = 2 * CH
NCHUNK = NWIN // CH

_mesh = plsc.VectorSubcoreMesh(core_axis_name="c", subcore_axis_name="s")


WD = 128
ND = EP // (NSUB * NCORE * WD)


@functools.partial(
    pl.kernel,
    mesh=_mesh,
    out_type=jax.ShapeDtypeStruct((NCORE, NP), jnp.float32),
    scratch_types=[
        pltpu.VMEM((2, WD), jnp.int32),
        pltpu.VMEM((WD,), jnp.float32),
        pltpu.VMEM((RPS,), jnp.float32),
        pltpu.VMEM_SHARED((NP,), jnp.float32),
        pltpu.SemaphoreType.DMA,
        pltpu.SemaphoreType.DMA,
        pltpu.SemaphoreType.DMA,
        pltpu.SemaphoreType.DMA,
    ],
)
def _deg_kernel(dstd_hbm, deg_hbm, idx_v, ones_v, init_v, acc_sh,
                isemA, isemB, ssemA, ssemB):
    c = lax.axis_index("c")
    s = lax.axis_index("s")
    w = s * NCORE + c
    for k in range(WD // 16):
        ones_v[pl.ds(k * 16, 16)] = jnp.full((16,), 1.0, jnp.float32)
    for k in range(RPS // 16):
        init_v[pl.ds(k * 16, 16)] = jnp.full((16,), 0.5, jnp.float32)
    isems = [isemA, isemB]
    ssems = [ssemA, ssemB]

    def i_start(j, p):
        pltpu.async_copy(dstd_hbm.at[w, j], idx_v.at[p], isems[p])

    def i_wait(j, p):
        pltpu.make_async_copy(dstd_hbm.at[w, j], idx_v.at[p],
                              isems[p]).wait()

    def s_start(p):
        pltpu.async_copy(ones_v, acc_sh.at[idx_v.at[p]], ssems[p], add=True)

    def s_wait(p):
        pltpu.make_async_copy(ones_v, acc_sh.at[idx_v.at[0]],
                              ssems[p]).wait()

    i_start(0, 0)
    i_start(1, 1)
    pltpu.sync_copy(init_v, acc_sh.at[pl.ds(s * RPS, RPS)])
    plsc.subcore_barrier()

    def body(t, carry):
        j0 = 2 * t
        for p in range(2):
            i_wait(j0 + p, p)
            s_start(p)

        @pl.when(t + 1 < ND // 2)
        def _():
            for p in range(2):
                s_wait(p)
                i_start(j0 + 2 + p, p)
        return carry

    lax.fori_loop(0, ND // 2, body, 0)
    for p in range(2):
        s_wait(p)
    plsc.subcore_barrier()
    pltpu.sync_copy(acc_sh.at[pl.ds(s * RPS, RPS)],
                    deg_hbm.at[c, pl.ds(s * RPS, RPS)])


@functools.partial(
    pl.kernel,
    mesh=_mesh,
    out_type=jax.ShapeDtypeStruct((NCORE, NP, HALF), jnp.float32),
    scratch_types=[
        pltpu.VMEM((CH, 2, W), jnp.int32),
        pltpu.VMEM((CH, 2, W), jnp.int32),
        pltpu.VMEM((CH, W), jnp.int32),
        pltpu.VMEM((CH, W), jnp.int32),
        pltpu.VMEM((NG, W, HALF), jnp.float32),
        pltpu.VMEM_SHARED((NP, HALF), jnp.float32),
        pltpu.SemaphoreType.DMA,
        pltpu.SemaphoreType.DMA,
        pltpu.SemaphoreType.DMA,
        pltpu.SemaphoreType.DMA,
        pltpu.SemaphoreType.DMA,
        pltpu.SemaphoreType.DMA,
    ],
)
def _msg_kernel(g_hbm, idx_hbm, m_hbm,
                ibuf0, ibuf1, sbuf0, sbuf1, rows_v, acc_sh,
                gsem0, gsem1, ssem0, ssem1, isem0, isem1):
    c = lax.axis_index("c")
    s = lax.axis_index("s")
    r0 = s * RPS

    def i_start(j0, ibuf, isem):
        pltpu.async_copy(idx_hbm.at[c, s, pl.ds(j0, CH)], ibuf, isem)

    def i_wait(j0, ibuf, isem):
        pltpu.make_async_copy(idx_hbm.at[c, s, pl.ds(j0, CH)], ibuf,
                              isem).wait()

    HW = W // 2

    def g_start(i, buf, ibuf, sem):
        pltpu.async_copy(g_hbm.at[ibuf.at[i, 0, pl.ds(0, HW)]],
                         rows_v.at[buf, pl.ds(0, HW)], sem)
        pltpu.async_copy(g_hbm.at[ibuf.at[i, 0, pl.ds(HW, HW)]],
                         rows_v.at[buf, pl.ds(HW, HW)], sem)

    def g_wait(i, buf, ibuf, sem):
        pltpu.make_async_copy(g_hbm.at[ibuf.at[i, 0, pl.ds(0, HW)]],
                              rows_v.at[buf, pl.ds(0, HW)], sem).wait()
        pltpu.make_async_copy(g_hbm.at[ibuf.at[i, 0, pl.ds(HW, HW)]],
                              rows_v.at[buf, pl.ds(HW, HW)], sem).wait()

    def dst_copy(ibuf, sbuf):
        for i in range(CH):
            for k in range(W // 16):
                sl = pl.ds(k * 16, 16)
                sbuf[i, sl] = ibuf[i, 1, sl]

    def s_start(i, buf, sbuf, sem):
        pltpu.async_copy(rows_v.at[buf], acc_sh.at[sbuf.at[i]], sem,
                         add=True)

    def s_wait(i, buf, sbuf, sem):
        pltpu.make_async_copy(rows_v.at[buf], acc_sh.at[sbuf.at[i]],
                              sem).wait()

    i_start(0, ibuf0, isem0)
    pltpu.sync_copy(g_hbm.at[pl.ds(c * NP + r0, RPS)],
                    acc_sh.at[pl.ds(r0, RPS)])
    i_wait(0, ibuf0, isem0)
    for i in range(CH):
        g_start(i, i, ibuf0, gsem0)
    i_start(CH, ibuf1, isem1)
    plsc.subcore_barrier()

    def pair(t, carry):
        a0 = (2 * t) * CH
        b0 = (2 * t + 1) * CH

        @pl.when(t > 0)
        def _():
            for i in range(CH):
                s_wait(i, CH + i, sbuf1, ssem1)
        i_wait(b0, ibuf1, isem1)
        for i in range(CH):
            g_start(i, CH + i, ibuf1, gsem1)
        for i in range(CH):
            g_wait(i, i, ibuf0, gsem0)
        dst_copy(ibuf0, sbuf0)
        for i in range(CH):
            s_start(i, i, sbuf0, ssem0)

        @pl.when(t + 1 < NCHUNK // 2)
        def _():
            i_start(a0 + 2 * CH, ibuf0, isem0)
        for i in range(CH):
            g_wait(i, CH + i, ibuf1, gsem1)
        dst_copy(ibuf1, sbuf1)
        for i in range(CH):
            s_wait(i, i, sbuf0, ssem0)

        @pl.when(t + 1 < NCHUNK // 2)
        def _():
            i_wait((2 * t + 2) * CH, ibuf0, isem0)
            for i in range(CH):
                g_start(i, i, ibuf0, gsem0)
            i_start((2 * t + 3) * CH, ibuf1, isem1)
        for i in range(CH):
            s_start(i, CH + i, sbuf1, ssem1)
        return carry

    lax.fori_loop(0, NCHUNK // 2, pair, 0)
    for i in range(CH):
        s_wait(i, CH + i, sbuf1, ssem1)
    plsc.subcore_barrier()
    pltpu.sync_copy(acc_sh.at[pl.ds(r0, RPS)],
                    m_hbm.at[c, pl.ds(r0, RPS)])


BM = 1000
GRID = N // BM


def _tc_first_body(x_ref, mean_ref, std_ref, w_ref, deg_ref, g_ref):
    h0 = (x_ref[...] - mean_ref[...]) / std_ref[...]
    p = jnp.dot(h0, w_ref[...], preferred_element_type=jnp.float32)
    dinv = lax.rsqrt(deg_ref[0] + deg_ref[1])
    g = p * dinv
    g_ref[0] = g[:, :HALF]
    g_ref[1] = g[:, HALF:]


def _tc_mid_body(m_ref, deg_ref, b_ref, w_ref, g_ref):
    dinv = lax.rsqrt(deg_ref[0] + deg_ref[1])
    mb = jnp.concatenate([m_ref[0], m_ref[1]], axis=1)
    h = jnp.maximum(mb * dinv + b_ref[...], 0.0)
    p = jnp.dot(h, w_ref[...], preferred_element_type=jnp.float32)
    g = p * dinv
    g_ref[0] = g[:, :HALF]
    g_ref[1] = g[:, HALF:]


def _tc_final_body(m_ref, deg_ref, b_ref, wl_ref, bl_ref, ostd_ref,
                   omean_ref, mask_ref, pq_ref, o_ref):
    dinv = lax.rsqrt(deg_ref[0] + deg_ref[1])
    mb = jnp.concatenate([m_ref[0], m_ref[1]], axis=1)
    h = mb * dinv + b_ref[...]
    o = jnp.dot(h, wl_ref[...], preferred_element_type=jnp.float32) + bl_ref[...]
    o = o * ostd_ref[...] + omean_ref[...]
    o_ref[...] = jnp.where(mask_ref[...], pq_ref[...], o)


def _row_spec(width):
    return pl.BlockSpec((BM, width), lambda i: (i, 0))


def _full_spec(shape):
    nd = len(shape)
    return pl.BlockSpec(shape, lambda i, _n=nd: (0,) * _n)


_deg_spec = pl.BlockSpec((NCORE, BM, 1), lambda i: (0, i, 0))
_gm_spec = pl.BlockSpec((NCORE, BM, HALF), lambda i: (0, i, 0))


def _tc_first(x_pad, in_mean, in_std, w1, deg3):
    return pl.pallas_call(
        _tc_first_body,
        grid=(GRID,),
        in_specs=[_row_spec(IN), _full_spec((1, IN)), _full_spec((1, IN)),
                  _full_spec((IN, H)), _deg_spec],
        out_specs=_gm_spec,
        out_shape=jax.ShapeDtypeStruct((NCORE, NP, HALF), jnp.float32),
    )(x_pad, in_mean.reshape(1, IN), in_std.reshape(1, IN), w1, deg3)


def _tc_mid(m, deg3, b, w):
    return pl.pallas_call(
        _tc_mid_body,
        grid=(GRID,),
        in_specs=[_gm_spec, _deg_spec, _full_spec((1, H)), _full_spec((H, H))],
        out_specs=_gm_spec,
        out_shape=jax.ShapeDtypeStruct((NCORE, NP, HALF), jnp.float32),
    )(m, deg3, b.reshape(1, H), w)


def _tc_final(m, deg3, b3, wlin, blin, out_std, out_mean, mask_pad, pq_pad):
    return pl.pallas_call(
        _tc_final_body,
        grid=(GRID,),
        in_specs=[_gm_spec, _deg_spec, _full_spec((1, H)), _full_spec((H, OUT)),
                  _full_spec((1, OUT)), _full_spec((1, OUT)), _full_spec((1, OUT)),
                  _row_spec(OUT), _row_spec(OUT)],
        out_specs=_row_spec(OUT),
        out_shape=jax.ShapeDtypeStruct((N, OUT), jnp.float32),
    )(m, deg3, b3.reshape(1, H), wlin, blin.reshape(1, OUT),
      out_std.reshape(1, OUT), out_mean.reshape(1, OUT), mask_pad, pq_pad)


def kernel(x, edge_index, PQVA_mask, PQVA_matrix, in_mean, in_std,
           out_mean, out_std, W1, b1, W2, b2, W3, b3, Wlin, blin):
    pad = EP - E
    pad_idx = N + (jnp.arange(pad, dtype=jnp.int32) % (NP - N))
    srcp = jnp.concatenate([edge_index[0], pad_idx])
    dstp = jnp.concatenate([edge_index[1], pad_idx])
    src3 = srcp.reshape(NSUB, NWIN, W)
    dst3 = dstp.reshape(NSUB, NWIN, W)
    idx_hbm = jnp.stack([
        jnp.stack([src3, dst3], axis=2),
        jnp.stack([src3 + NP, dst3], axis=2),
    ])
    dstd = dstp.reshape(NSUB * NCORE, ND, WD)

    deg3 = _deg_kernel(dst3).reshape(NCORE, NP, 1)

    g1 = _tc_first(x, in_mean, in_std, W1, deg3)
    m1 = _msg_kernel(g1.reshape(NCORE * NP, HALF), src3b, dst3)
    g2 = _tc_mid(m1, deg3, b1, W2)
    m2 = _msg_kernel(g2.reshape(NCORE * NP, HALF), src3b, dst3)
    g3 = _tc_mid(m2, deg3, b2, W3)
    m3 = _msg_kernel(g3.reshape(NCORE * NP, HALF), src3b, dst3)
    return _tc_final(m3, deg3, b3, Wlin, blin, out_std, out_mean,
                     PQVA_mask, PQVA_matrix)


# speedup vs baseline: 20.2067x; 1.0005x over previous
import functools

import jax
import jax.numpy as jnp
from jax import lax
from jax.experimental import pallas as pl
from jax.experimental.pallas import tpu as pltpu
from jax.experimental.pallas import tpu_sc as plsc

N = 10000
NP = 10240
E = 320000
IN = 128
H = 256
OUT = 128
HALF = 128

NSUB = 16
NCORE = 2
W = 128
NWIN = 160
EP = NSUB * NWIN * W
RPS = NP // NSUB
CH = 1
NG = 2 * CH
NCHUNK = NWIN // CH

_mesh = plsc.VectorSubcoreMesh(core_axis_name="c", subcore_axis_name="s")


WD = 128
ND = EP // (NSUB * NCORE * WD)


@functools.partial(
    pl.kernel,
    mesh=_mesh,
    out_type=jax.ShapeDtypeStruct((NCORE, NP), jnp.float32),
    scratch_types=[
        pltpu.VMEM((2, WD), jnp.int32),
        pltpu.VMEM((WD,), jnp.float32),
        pltpu.VMEM((RPS,), jnp.float32),
        pltpu.VMEM_SHARED((NP,), jnp.float32),
        pltpu.SemaphoreType.DMA,
        pltpu.SemaphoreType.DMA,
        pltpu.SemaphoreType.DMA,
        pltpu.SemaphoreType.DMA,
    ],
)
def _deg_kernel(dstd_hbm, deg_hbm, idx_v, ones_v, init_v, acc_sh,
                isemA, isemB, ssemA, ssemB):
    c = lax.axis_index("c")
    s = lax.axis_index("s")
    w = s * NCORE + c
    for k in range(WD // 16):
        ones_v[pl.ds(k * 16, 16)] = jnp.full((16,), 1.0, jnp.float32)
    for k in range(RPS // 16):
        init_v[pl.ds(k * 16, 16)] = jnp.full((16,), 0.5, jnp.float32)
    isems = [isemA, isemB]
    ssems = [ssemA, ssemB]

    def i_start(j, p):
        pltpu.async_copy(dstd_hbm.at[w, j], idx_v.at[p], isems[p])

    def i_wait(j, p):
        pltpu.make_async_copy(dstd_hbm.at[w, j], idx_v.at[p],
                              isems[p]).wait()

    def s_start(p):
        pltpu.async_copy(ones_v, acc_sh.at[idx_v.at[p]], ssems[p], add=True)

    def s_wait(p):
        pltpu.make_async_copy(ones_v, acc_sh.at[idx_v.at[0]],
                              ssems[p]).wait()

    i_start(0, 0)
    i_start(1, 1)
    pltpu.sync_copy(init_v, acc_sh.at[pl.ds(s * RPS, RPS)])
    plsc.subcore_barrier()

    def body(t, carry):
        j0 = 2 * t
        for p in range(2):
            i_wait(j0 + p, p)
            s_start(p)

        @pl.when(t + 1 < ND // 2)
        def _():
            for p in range(2):
                s_wait(p)
                i_start(j0 + 2 + p, p)
        return carry

    lax.fori_loop(0, ND // 2, body, 0)
    for p in range(2):
        s_wait(p)
    plsc.subcore_barrier()
    pltpu.sync_copy(acc_sh.at[pl.ds(s * RPS, RPS)],
                    deg_hbm.at[c, pl.ds(s * RPS, RPS)])


@functools.partial(
    pl.kernel,
    mesh=_mesh,
    out_type=jax.ShapeDtypeStruct((NCORE, NP, HALF), jnp.float32),
    scratch_types=[
        pltpu.VMEM((CH, 2, W), jnp.int32),
        pltpu.VMEM((CH, 2, W), jnp.int32),
        pltpu.VMEM((CH, W), jnp.int32),
        pltpu.VMEM((CH, W), jnp.int32),
        pltpu.VMEM((NG, W, HALF), jnp.float32),
        pltpu.VMEM_SHARED((NP, HALF), jnp.float32),
        pltpu.SemaphoreType.DMA,
        pltpu.SemaphoreType.DMA,
        pltpu.SemaphoreType.DMA,
        pltpu.SemaphoreType.DMA,
        pltpu.SemaphoreType.DMA,
        pltpu.SemaphoreType.DMA,
    ],
)
def _msg_kernel(g_hbm, idx_hbm, m_hbm,
                ibuf0, ibuf1, sbuf0, sbuf1, rows_v, acc_sh,
                gsem0, gsem1, ssem0, ssem1, isem0, isem1):
    c = lax.axis_index("c")
    s = lax.axis_index("s")
    r0 = s * RPS

    def i_start(j0, ibuf, isem):
        pltpu.async_copy(idx_hbm.at[c, s, pl.ds(j0, CH)], ibuf, isem)

    def i_wait(j0, ibuf, isem):
        pltpu.make_async_copy(idx_hbm.at[c, s, pl.ds(j0, CH)], ibuf,
                              isem).wait()

    HW = W // 2

    def g_start(i, buf, ibuf, sem):
        pltpu.async_copy(g_hbm.at[ibuf.at[i, 0, pl.ds(0, HW)]],
                         rows_v.at[buf, pl.ds(0, HW)], sem)
        pltpu.async_copy(g_hbm.at[ibuf.at[i, 0, pl.ds(HW, HW)]],
                         rows_v.at[buf, pl.ds(HW, HW)], sem)

    def g_wait(i, buf, ibuf, sem):
        pltpu.make_async_copy(g_hbm.at[ibuf.at[i, 0, pl.ds(0, HW)]],
                              rows_v.at[buf, pl.ds(0, HW)], sem).wait()
        pltpu.make_async_copy(g_hbm.at[ibuf.at[i, 0, pl.ds(HW, HW)]],
                              rows_v.at[buf, pl.ds(HW, HW)], sem).wait()

    def dst_copy(ibuf, sbuf):
        for i in range(CH):
            for k in range(W // 16):
                sl = pl.ds(k * 16, 16)
                sbuf[i, sl] = ibuf[i, 1, sl]

    def s_start(i, buf, sbuf, sem):
        pltpu.async_copy(rows_v.at[buf], acc_sh.at[sbuf.at[i]], sem,
                         add=True)

    def s_wait(i, buf, sbuf, sem):
        pltpu.make_async_copy(rows_v.at[buf], acc_sh.at[sbuf.at[i]],
                              sem).wait()

    i_start(0, ibuf0, isem0)
    pltpu.sync_copy(g_hbm.at[pl.ds(c * NP + r0, RPS)],
                    acc_sh.at[pl.ds(r0, RPS)])
    i_wait(0, ibuf0, isem0)
    for i in range(CH):
        g_start(i, i, ibuf0, gsem0)
    i_start(CH, ibuf1, isem1)
    plsc.subcore_barrier()

    def pair(t, carry):
        a0 = (2 * t) * CH
        b0 = (2 * t + 1) * CH

        @pl.when(t > 0)
        def _():
            for i in range(CH):
                s_wait(i, CH + i, sbuf1, ssem1)
        i_wait(b0, ibuf1, isem1)
        for i in range(CH):
            g_start(i, CH + i, ibuf1, gsem1)
        for i in range(CH):
            g_wait(i, i, ibuf0, gsem0)
        dst_copy(ibuf0, sbuf0)
        for i in range(CH):
            s_start(i, i, sbuf0, ssem0)

        @pl.when(t + 1 < NCHUNK // 2)
        def _():
            i_start(a0 + 2 * CH, ibuf0, isem0)
        for i in range(CH):
            g_wait(i, CH + i, ibuf1, gsem1)
        dst_copy(ibuf1, sbuf1)
        for i in range(CH):
            s_wait(i, i, sbuf0, ssem0)

        @pl.when(t + 1 < NCHUNK // 2)
        def _():
            i_wait((2 * t + 2) * CH, ibuf0, isem0)
            for i in range(CH):
                g_start(i, i, ibuf0, gsem0)
            i_start((2 * t + 3) * CH, ibuf1, isem1)
        for i in range(CH):
            s_start(i, CH + i, sbuf1, ssem1)
        return carry

    lax.fori_loop(0, NCHUNK // 2, pair, 0)
    for i in range(CH):
        s_wait(i, CH + i, sbuf1, ssem1)
    plsc.subcore_barrier()
    pltpu.sync_copy(acc_sh.at[pl.ds(r0, RPS)],
                    m_hbm.at[c, pl.ds(r0, RPS)])


BM = 1000
GRID = N // BM


def _tc_first_body(x_ref, mean_ref, std_ref, w_ref, deg_ref, g_ref):
    h0 = (x_ref[...] - mean_ref[...]) / std_ref[...]
    p = jnp.dot(h0, w_ref[...], preferred_element_type=jnp.float32)
    dinv = lax.rsqrt(deg_ref[0] + deg_ref[1])
    g = p * dinv
    g_ref[0] = g[:, :HALF]
    g_ref[1] = g[:, HALF:]


def _tc_mid_body(m_ref, deg_ref, b_ref, w_ref, g_ref):
    dinv = lax.rsqrt(deg_ref[0] + deg_ref[1])
    mb = jnp.concatenate([m_ref[0], m_ref[1]], axis=1)
    h = jnp.maximum(mb * dinv + b_ref[...], 0.0)
    p = jnp.dot(h, w_ref[...], preferred_element_type=jnp.float32)
    g = p * dinv
    g_ref[0] = g[:, :HALF]
    g_ref[1] = g[:, HALF:]


def _tc_final_body(m_ref, deg_ref, b_ref, wl_ref, bl_ref, ostd_ref,
                   omean_ref, mask_ref, pq_ref, o_ref):
    dinv = lax.rsqrt(deg_ref[0] + deg_ref[1])
    mb = jnp.concatenate([m_ref[0], m_ref[1]], axis=1)
    h = mb * dinv + b_ref[...]
    o = jnp.dot(h, wl_ref[...], preferred_element_type=jnp.float32) + bl_ref[...]
    o = o * ostd_ref[...] + omean_ref[...]
    o_ref[...] = jnp.where(mask_ref[...], pq_ref[...], o)


def _row_spec(width):
    return pl.BlockSpec((BM, width), lambda i: (i, 0))


def _full_spec(shape):
    nd = len(shape)
    return pl.BlockSpec(shape, lambda i, _n=nd: (0,) * _n)


_deg_spec = pl.BlockSpec((NCORE, BM, 1), lambda i: (0, i, 0))
_gm_spec = pl.BlockSpec((NCORE, BM, HALF), lambda i: (0, i, 0))


def _tc_first(x_pad, in_mean, in_std, w1, deg3):
    return pl.pallas_call(
        _tc_first_body,
        grid=(GRID,),
        in_specs=[_row_spec(IN), _full_spec((1, IN)), _full_spec((1, IN)),
                  _full_spec((IN, H)), _deg_spec],
        out_specs=_gm_spec,
        out_shape=jax.ShapeDtypeStruct((NCORE, NP, HALF), jnp.float32),
    )(x_pad, in_mean.reshape(1, IN), in_std.reshape(1, IN), w1, deg3)


def _tc_mid(m, deg3, b, w):
    return pl.pallas_call(
        _tc_mid_body,
        grid=(GRID,),
        in_specs=[_gm_spec, _deg_spec, _full_spec((1, H)), _full_spec((H, H))],
        out_specs=_gm_spec,
        out_shape=jax.ShapeDtypeStruct((NCORE, NP, HALF), jnp.float32),
    )(m, deg3, b.reshape(1, H), w)


def _tc_final(m, deg3, b3, wlin, blin, out_std, out_mean, mask_pad, pq_pad):
    return pl.pallas_call(
        _tc_final_body,
        grid=(GRID,),
        in_specs=[_gm_spec, _deg_spec, _full_spec((1, H)), _full_spec((H, OUT)),
                  _full_spec((1, OUT)), _full_spec((1, OUT)), _full_spec((1, OUT)),
                  _row_spec(OUT), _row_spec(OUT)],
        out_specs=_row_spec(OUT),
        out_shape=jax.ShapeDtypeStruct((N, OUT), jnp.float32),
    )(m, deg3, b3.reshape(1, H), wlin, blin.reshape(1, OUT),
      out_std.reshape(1, OUT), out_mean.reshape(1, OUT), mask_pad, pq_pad)


def kernel(x, edge_index, PQVA_mask, PQVA_matrix, in_mean, in_std,
           out_mean, out_std, W1, b1, W2, b2, W3, b3, Wlin, blin):
    pad = EP - E
    pad_idx = N + (jnp.arange(pad, dtype=jnp.int32) % (NP - N))
    srcp = jnp.concatenate([edge_index[0], pad_idx])
    dstp = jnp.concatenate([edge_index[1], pad_idx])
    src3 = srcp.reshape(NSUB, NWIN, W)
    dst3 = dstp.reshape(NSUB, NWIN, W)
    idx_hbm = jnp.stack([
        jnp.stack([src3, dst3], axis=2),
        jnp.stack([src3 + NP, dst3], axis=2),
    ])
    dstd = dstp.reshape(NSUB * NCORE, ND, WD)

    deg3 = _deg_kernel(dstd).reshape(NCORE, NP, 1)

    g1 = _tc_first(x, in_mean, in_std, W1, deg3)
    m1 = _msg_kernel(g1.reshape(NCORE * NP, HALF), idx_hbm)
    g2 = _tc_mid(m1, deg3, b1, W2)
    m2 = _msg_kernel(g2.reshape(NCORE * NP, HALF), idx_hbm)
    g3 = _tc_mid(m2, deg3, b2, W3)
    m3 = _msg_kernel(g3.reshape(NCORE * NP, HALF), idx_hbm)
    return _tc_final(m3, deg3, b3, Wlin, blin, out_std, out_mean,
                     PQVA_mask, PQVA_matrix)
